# baseline (speedup 1.0000x reference)
import functools

import jax
import jax.numpy as jnp
from jax import lax
from jax.experimental import pallas as pl
from jax.experimental.pallas import tpu as pltpu
from jax.experimental.pallas import tpu_sc as plsc

_VOCAB = 32000
_SRC = 1024
_W = _VOCAB + _SRC
_N = 4096
_EPS = 1e-20

_NC = 2
_NS = 16
_NW = _NC * _NS
_T = _N // _NW
_L = 16


_TPR = _W // 128

_SQRT2 = 1.4142135
_C3 = 1.0 / 3.0
_C5 = 0.2
_C7 = 1.0 / 7.0
_TWO_OVER_LN2 = 2.8853900817779268
_LN2 = 0.6931471805599453


def _sc_probs(scores, target, align, mask):
    mesh = plsc.VectorSubcoreMesh(core_axis_name="c", subcore_axis_name="s")

    @functools.partial(
        pl.kernel,
        mesh=mesh,
        out_type=jax.ShapeDtypeStruct((_NW, _L), jnp.float32),
        scratch_types=[
            pltpu.VMEM((_T,), jnp.int32),
            pltpu.VMEM((_T,), jnp.int32),
            pltpu.VMEM((_T,), jnp.int32),
            pltpu.VMEM((_T,), jnp.int32),
            pltpu.VMEM((_T,), jnp.int32),
            pltpu.VMEM((_T,), jnp.float32),
            pltpu.VMEM((_T,), jnp.float32),
            pltpu.VMEM((_L,), jnp.float32),
            pltpu.SemaphoreType.DMA,
        ],
    )
    def k(scores_hbm, tgt_hbm, align_hbm, mask_hbm, out_hbm,
          tgt_v, al_v, m_v, ix1_v, ix2_v, vp_v, cp_v, p_v, sem):
        wid = lax.axis_index("s") * _NC + lax.axis_index("c")
        base = wid * _T
        s1 = pltpu.async_copy(tgt_hbm.at[pl.ds(base, _T)], tgt_v, sem)
        s2 = pltpu.async_copy(align_hbm.at[pl.ds(base, _T)], al_v, sem)
        s3 = pltpu.async_copy(mask_hbm.at[pl.ds(base, _T)], m_v, sem)
        s1.wait()
        s2.wait()
        s3.wait()
        lanes = lax.iota(jnp.int32, _L)
        r0 = (wid >> 2) * 512 + (wid & 3)
        for j in range(_T // _L):
            sl = pl.ds(j * _L, _L)
            r = r0 + 4 * (j * _L + lanes)
            tile_base = (r >> 3) * (_TPR * 1024) + (r & 7) * 128
            c1 = tgt_v[sl]
            ix1_v[sl] = tile_base + (c1 >> 7) * 1024 + (c1 & 127)
            c2 = _VOCAB + al_v[sl]
            ix2_v[sl] = tile_base + (c2 >> 7) * 1024 + (c2 & 127)
        g1 = pltpu.async_copy(scores_hbm.at[ix1_v], vp_v, sem)
        g2 = pltpu.async_copy(scores_hbm.at[ix2_v], cp_v, sem)
        g1.wait()
        g2.wait()
        acc = jnp.zeros((_L,), jnp.float32)
        for j in range(_T // _L):
            sl = pl.ds(j * _L, _L)
            t = tgt_v[sl]
            a = al_v[sl]
            no_align = a == 0
            cp = jnp.where(no_align, 0.0, cp_v[sl]) + _EPS
            non_copy = no_align | (t != 0)
            p = jnp.where(non_copy, cp + vp_v[sl], cp)
            p = jnp.where(m_v[sl] == 1, 1.0, p)
            x = lax.bitcast_convert_type(p, jnp.int32)
            e = (x >> 23) - 127
            m = lax.bitcast_convert_type((x & 0x007FFFFF) | 0x3F800000, jnp.float32)
            big = m >= _SQRT2
            m = jnp.where(big, m * 0.5, m)
            e = jnp.where(big, e + 1, e)
            s = (m - 1.0) / (m + 1.0)
            s2 = s * s
            at = s * (1.0 + s2 * (_C3 + s2 * (_C5 + s2 * _C7)))
            acc = acc + (e.astype(jnp.float32) + at * _TWO_OVER_LN2)
        p_v[...] = acc
        pltpu.sync_copy(p_v, out_hbm.at[wid])

    return k(scores, target, align, mask)


def _tc_loss(partials):
    def body(p_ref, o_ref):
        o_ref[...] = ((-_LN2) * jnp.sum(p_ref[...])).reshape(1, 1)

    out = pl.pallas_call(
        body,
        out_shape=jax.ShapeDtypeStruct((1, 1), jnp.float32),
    )(partials)
    return out[0, 0]


def kernel(scores, target, align, tgt_loss_mask):
    flat = (
        scores.reshape(_N // 8, 8, _TPR, 128)
        .transpose(0, 2, 1, 3)
        .reshape(-1)
    )

    def _phys(a):
        return (
            a.astype(jnp.int32).reshape(8, 128, 4).transpose(0, 2, 1).reshape(-1)
        )

    probs = _sc_probs(flat, _phys(target), _phys(align), _phys(tgt_loss_mask))
    return (-_LN2) * jnp.sum(probs)


# speedup vs baseline: 1.0461x; 1.0461x over previous
import functools

import jax
import jax.numpy as jnp
from jax import lax
from jax.experimental import pallas as pl
from jax.experimental.pallas import tpu as pltpu
from jax.experimental.pallas import tpu_sc as plsc

_VOCAB = 32000
_SRC = 1024
_W = _VOCAB + _SRC
_N = 4096
_EPS = 1e-20

_NC = 2
_NS = 16
_NW = _NC * _NS
_T = _N // _NW
_L = 16


_TPR = _W // 128

_SQRT2 = 1.4142135
_C3 = 1.0 / 3.0
_C5 = 0.2
_C7 = 1.0 / 7.0
_TWO_OVER_LN2 = 2.8853900817779268
_LN2 = 0.6931471805599453


def _sc_probs(scores, target, align, mask):
    mesh = plsc.VectorSubcoreMesh(core_axis_name="c", subcore_axis_name="s")

    @functools.partial(
        pl.kernel,
        mesh=mesh,
        out_type=jax.ShapeDtypeStruct((_NW, _L), jnp.float32),
        scratch_types=[
            pltpu.VMEM((_T,), jnp.int32),
            pltpu.VMEM((_T,), jnp.int32),
            pltpu.VMEM((_T,), jnp.int32),
            pltpu.VMEM((_T,), jnp.int32),
            pltpu.VMEM((_T,), jnp.int32),
            pltpu.VMEM((_T,), jnp.float32),
            pltpu.VMEM((_T,), jnp.float32),
            pltpu.VMEM((_L,), jnp.float32),
            pltpu.SemaphoreType.DMA,
        ],
    )
    def k(scores_hbm, tgt_hbm, align_hbm, mask_hbm, out_hbm,
          tgt_v, al_v, m_v, ix1_v, ix2_v, vp_v, cp_v, p_v, sem):
        wid = lax.axis_index("s") * _NC + lax.axis_index("c")
        base = wid * _T
        s1 = pltpu.async_copy(tgt_hbm.at[pl.ds(base, _T)], tgt_v, sem)
        s2 = pltpu.async_copy(align_hbm.at[pl.ds(base, _T)], al_v, sem)
        s3 = pltpu.async_copy(mask_hbm.at[pl.ds(base, _T)], m_v, sem)
        s1.wait()
        s2.wait()
        lanes = lax.iota(jnp.int32, _L)
        r0 = (wid >> 2) * 512 + (wid & 3)
        for j in range(_T // _L):
            sl = pl.ds(j * _L, _L)
            r = r0 + 4 * (j * _L + lanes)
            tile_base = (r >> 3) * (_TPR * 1024) + (r & 7) * 128
            c1 = tgt_v[sl]
            ix1_v[sl] = tile_base + (c1 >> 7) * 1024 + (c1 & 127)
            c2 = _VOCAB + al_v[sl]
            ix2_v[sl] = tile_base + (c2 >> 7) * 1024 + (c2 & 127)
        g1 = pltpu.async_copy(scores_hbm.at[ix1_v], vp_v, sem)
        g2 = pltpu.async_copy(scores_hbm.at[ix2_v], cp_v, sem)
        s3.wait()
        g1.wait()
        g2.wait()
        acc = jnp.zeros((_L,), jnp.float32)
        for j in range(_T // _L):
            sl = pl.ds(j * _L, _L)
            t = tgt_v[sl]
            a = al_v[sl]
            no_align = a == 0
            cp = jnp.where(no_align, 0.0, cp_v[sl]) + _EPS
            non_copy = no_align | (t != 0)
            p = jnp.where(non_copy, cp + vp_v[sl], cp)
            p = jnp.where(m_v[sl] == 1, 1.0, p)
            x = lax.bitcast_convert_type(p, jnp.int32)
            e = (x >> 23) - 127
            m = lax.bitcast_convert_type((x & 0x007FFFFF) | 0x3F800000, jnp.float32)
            big = m >= _SQRT2
            m = jnp.where(big, m * 0.5, m)
            e = jnp.where(big, e + 1, e)
            s = (m - 1.0) / (m + 1.0)
            s2 = s * s
            at = s * (1.0 + s2 * (_C3 + s2 * (_C5 + s2 * _C7)))
            acc = acc + (e.astype(jnp.float32) + at * _TWO_OVER_LN2)
        p_v[...] = acc
        pltpu.sync_copy(p_v, out_hbm.at[wid])

    return k(scores, target, align, mask)


def _tc_loss(partials):
    def body(p_ref, o_ref):
        o_ref[...] = ((-_LN2) * jnp.sum(p_ref[...])).reshape(1, 1)

    out = pl.pallas_call(
        body,
        out_shape=jax.ShapeDtypeStruct((1, 1), jnp.float32),
    )(partials)
    return out[0, 0]


def kernel(scores, target, align, tgt_loss_mask):
    flat = (
        scores.reshape(_N // 8, 8, _TPR, 128)
        .transpose(0, 2, 1, 3)
        .reshape(-1)
    )

    def _phys(a):
        return (
            a.astype(jnp.int32).reshape(8, 128, 4).transpose(0, 2, 1).reshape(-1)
        )

    probs = _sc_probs(flat, _phys(target), _phys(align), _phys(tgt_loss_mask))
    return _tc_loss(probs)
